# baseline (speedup 1.0000x reference)
import functools

import jax
import jax.numpy as jnp
from jax import lax
from jax.experimental import pallas as pl
from jax.experimental.pallas import tpu as pltpu
from jax.experimental.pallas import tpu_sc as plsc

N_TABLES = 26
VOCAB = 100000
DIM = 32
BATCH = 16384
OUT_DIM = 5

VB = 1024
NVB = 98
SROWS = NVB * VB // 4

NW = 32
BW = BATCH // NW
CHUNK = 128
NCH = N_TABLES * BW // CHUNK
GRP = CHUNK // 16
NBUF = 2
DB = 4


def _tc_repack(tab_t):
    def body(in_ref, out_ref):
        x = in_ref[0]
        for m in range(4):
            out_ref[0, :, 32 * m:32 * (m + 1)] = x[:, 256 * m:256 * (m + 1)].T

    return pl.pallas_call(
        body,
        grid=(N_TABLES, NVB),
        in_specs=[pl.BlockSpec((1, DIM, VB), lambda j, v: (j, 0, v))],
        out_specs=pl.BlockSpec((1, VB // 4, 128), lambda j, v: (j, v, 0)),
        out_shape=jax.ShapeDtypeStruct((N_TABLES, SROWS, 128), jnp.float32),
        compiler_params=pltpu.CompilerParams(
            dimension_semantics=("parallel", "arbitrary")),
    )(tab_t)


def _sc_fused(tab128, idx_flat, wbc):
    mesh = plsc.VectorSubcoreMesh(core_axis_name="c", subcore_axis_name="s")
    per_w = N_TABLES * BW

    @functools.partial(
        pl.kernel,
        out_type=jax.ShapeDtypeStruct((OUT_DIM * BATCH,), jnp.float32),
        mesh=mesh,
        scratch_types=[
            pltpu.VMEM((per_w,), jnp.int32),
            pltpu.VMEM((NCH * CHUNK,), jnp.int32),
            pltpu.VMEM((NBUF, CHUNK, 128), jnp.float32),
            pltpu.VMEM((N_TABLES * DIM * OUT_DIM * 16,), jnp.float32),
            pltpu.VMEM((OUT_DIM * BW,), jnp.float32),
            pltpu.SemaphoreType.DMA,
            pltpu.SemaphoreType.DMA,
        ],
        compiler_params=pltpu.CompilerParams(use_tc_tiling_on_sc=False,
                                             needs_layout_passes=False),
    )
    def k(tab_hbm, idx_hbm, w_hbm, out_hbm, idx_v, sidx_v, bufs, w_v,
          acc_v, *sems):
        wid = lax.axis_index("s") * 2 + lax.axis_index("c")
        pltpu.sync_copy(idx_hbm.at[pl.ds(wid * per_w, per_w)], idx_v)
        pltpu.sync_copy(w_hbm, w_v)

        @pl.loop(0, NCH)
        def _(c):
            j = c // 4
            base = c * CHUNK
            for q in range(GRP):
                v = idx_v[pl.ds(base + q * 16, 16)]
                srow = (lax.shift_right_logical(v, 10) * 256
                        + (v & 255) + j * SROWS)
                sidx_v[pl.ds(base + q * 16, 16)] = srow

        zeros = jnp.zeros((16,), jnp.float32)

        @pl.loop(0, OUT_DIM * BW // 16)
        def _(g):
            acc_v[pl.ds(g * 16, 16)] = zeros

        def gather(c, bslot, sem):
            pltpu.async_copy(tab_hbm.at[sidx_v.at[pl.ds(c * CHUNK, CHUNK)]],
                             bufs.at[bslot], sem)

        def wait_gather(bslot, sem):
            pltpu.make_async_copy(tab_hbm.at[sidx_v.at[pl.ds(0, CHUNK)]],
                                  bufs.at[bslot], sem).wait()

        lane_iota = lax.iota(jnp.int32, 16)

        def compute(c, bslot):
            j = c // 4
            base = c * CHUNK
            bbase = (c % 4) * CHUNK
            for db in range(DIM // DB):
                wv = [w_v[pl.ds((j * 160 + (db * DB + dd) * 5 + o) * 16, 16)]
                      for dd in range(DB) for o in range(OUT_DIM)]

                @pl.loop(0, GRP)
                def _(q):
                    v = idx_v[pl.ds(base + q * 16, 16)]
                    lanes0 = (lax.shift_right_logical(v, 8) & 3) * DIM + db * DB
                    rows = q * 16 + lane_iota
                    acc = [acc_v[pl.ds(o * BW + bbase + q * 16, 16)]
                           for o in range(OUT_DIM)]
                    for dd in range(DB):
                        col = plsc.load_gather(bufs.at[bslot],
                                               [rows, lanes0 + dd])
                        for o in range(OUT_DIM):
                            acc[o] = acc[o] + col * wv[dd * OUT_DIM + o]
                    for o in range(OUT_DIM):
                        acc_v[pl.ds(o * BW + bbase + q * 16, 16)] = acc[o]

        for i in range(NBUF):
            gather(i, i, sems[i])

        @pl.loop(0, NCH // NBUF)
        def _(t):
            for i in range(NBUF):
                c = t * NBUF + i
                wait_gather(i, sems[i])
                compute(c, i)

                @pl.when(c + NBUF < NCH)
                def _():
                    gather(c + NBUF, i, sems[i])

        for o in range(OUT_DIM):
            pltpu.sync_copy(
                acc_v.at[pl.ds(o * BW, BW)],
                out_hbm.at[pl.ds(o * BATCH + wid * BW, BW)])

    return k(tab128, idx_flat, wbc)


def kernel(inputs, tables, W, b):
    tab_t = jnp.transpose(tables, (0, 2, 1))
    tab128 = _tc_repack(tab_t).reshape(N_TABLES * SROWS, 128)
    idx_flat = jnp.transpose(inputs.reshape(N_TABLES, NW, BW),
                             (1, 0, 2)).reshape(-1)
    wbc = jnp.broadcast_to(W.reshape(-1)[:, None], (N_TABLES * DIM * OUT_DIM, 16)).reshape(-1)
    out1 = _sc_fused(tab128, idx_flat, wbc)
    return out1.reshape(OUT_DIM, BATCH).T + b


# speedup vs baseline: 1.3224x; 1.3224x over previous
import functools

import jax
import jax.numpy as jnp
from jax import lax
from jax.experimental import pallas as pl
from jax.experimental.pallas import tpu as pltpu
from jax.experimental.pallas import tpu_sc as plsc

N_TABLES = 26
VOCAB = 100000
DIM = 32
BATCH = 16384
OUT_DIM = 5

TOT_ROWS = N_TABLES * BATCH
NW = 32
ROWS_PER_W = TOT_ROWS // NW
CHUNK = 128
CHUNKS_PER_W = ROWS_PER_W // CHUNK
NBUF = 4
ROUNDS = CHUNKS_PER_W // NBUF


def _sc_gather(tab_flat, gidx):
    mesh = plsc.VectorSubcoreMesh(core_axis_name="c", subcore_axis_name="s")

    @functools.partial(
        pl.kernel,
        out_type=jax.ShapeDtypeStruct((TOT_ROWS, DIM), jnp.float32),
        mesh=mesh,
        scratch_types=[
            pltpu.VMEM((CHUNKS_PER_W, CHUNK), jnp.int32),
            pltpu.VMEM((NBUF, CHUNK, DIM), jnp.float32),
            pltpu.VMEM((NBUF, CHUNK, DIM), jnp.float32),
            pltpu.SemaphoreType.DMA,
            pltpu.SemaphoreType.DMA,
            pltpu.SemaphoreType.DMA,
            pltpu.SemaphoreType.DMA,
        ],
        compiler_params=pltpu.CompilerParams(use_tc_tiling_on_sc=False),
    )
    def k(tab_hbm, idx_hbm, out_hbm, idx_v, buf_a, buf_b, sga, sgb, swa, swb):
        wid = lax.axis_index("s") * 2 + lax.axis_index("c")
        c0 = wid * CHUNKS_PER_W
        r0 = c0 * CHUNK

        pltpu.sync_copy(idx_hbm.at[pl.ds(c0, CHUNKS_PER_W)], idx_v)

        def gather_round(r, buf, sem):
            for i in range(NBUF):
                pltpu.async_copy(tab_hbm.at[idx_v.at[r * NBUF + i]],
                                 buf.at[i], sem)

        def write_round(r, buf, sem):
            for i in range(NBUF):
                pltpu.async_copy(buf.at[i],
                                 out_hbm.at[pl.ds(r0 + (r * NBUF + i) * CHUNK,
                                                  CHUNK)], sem)

        def wait_gathers(buf, sem):
            for i in range(NBUF):
                pltpu.make_async_copy(tab_hbm.at[idx_v.at[0]],
                                      buf.at[i], sem).wait()

        def wait_writes(buf, sem):
            for i in range(NBUF):
                pltpu.make_async_copy(buf.at[i],
                                      out_hbm.at[pl.ds(0, CHUNK)], sem).wait()

        gather_round(0, buf_a, sga)

        @pl.loop(0, ROUNDS // 2)
        def _(t):
            @pl.when(t > 0)
            def _():
                wait_writes(buf_b, swb)
            gather_round(2 * t + 1, buf_b, sgb)
            wait_gathers(buf_a, sga)
            write_round(2 * t, buf_a, swa)
            wait_gathers(buf_b, sgb)
            wait_writes(buf_a, swa)

            @pl.when(t < ROUNDS // 2 - 1)
            def _():
                gather_round(2 * t + 2, buf_a, sga)
            write_round(2 * t + 1, buf_b, swb)

        wait_writes(buf_b, swb)

    return k(tab_flat, gidx)


BB = 512


def _tc_dense(emb3, w3, b2):
    def body(emb_ref, w_ref, b_ref, out_ref):
        acc = jnp.zeros((BB, OUT_DIM), jnp.float32)
        for j in range(N_TABLES):
            acc = acc + jnp.dot(emb_ref[j], w_ref[j],
                                precision=jax.lax.Precision.HIGHEST,
                                preferred_element_type=jnp.float32)
        out_ref[...] = acc + b_ref[...]

    return pl.pallas_call(
        body,
        grid=(BATCH // BB,),
        in_specs=[
            pl.BlockSpec((N_TABLES, BB, DIM), lambda i: (0, i, 0)),
            pl.BlockSpec((N_TABLES, DIM, OUT_DIM), lambda i: (0, 0, 0)),
            pl.BlockSpec((1, OUT_DIM), lambda i: (0, 0)),
        ],
        out_specs=pl.BlockSpec((BB, OUT_DIM), lambda i: (i, 0)),
        out_shape=jax.ShapeDtypeStruct((BATCH, OUT_DIM), jnp.float32),
        compiler_params=pltpu.CompilerParams(
            dimension_semantics=("parallel",)),
    )(emb3, w3, b2)


def kernel(inputs, tables, W, b):
    offs = (jnp.arange(N_TABLES, dtype=jnp.int32) * VOCAB)[:, None]
    gidx = (inputs + offs).reshape(TOT_ROWS // CHUNK, CHUNK)
    tab_flat = tables.reshape(N_TABLES * VOCAB, DIM)
    emb = _sc_gather(tab_flat, gidx)
    return _tc_dense(emb.reshape(N_TABLES, BATCH, DIM),
                     W.reshape(N_TABLES, DIM, OUT_DIM),
                     b.reshape(1, OUT_DIM))
